# Initial kernel scaffold; baseline (speedup 1.0000x reference)
import jax
import jax.numpy as jnp
from jax.experimental import pallas as pl


def kernel(x1, x2, edge_index1, edge_index2, batch1, batch2, W1a, b1a, W1b, b1b, W2a, b2a, W2b, b2b, Wl1, bl1, Wl2, bl2):
    raise NotImplementedError("write your pallas kernel here")


# speedup vs baseline: 4.7557x; 4.7557x over previous
import functools

import jax
import jax.numpy as jnp
from jax import lax
from jax.experimental import pallas as pl
from jax.experimental.pallas import tpu as pltpu
from jax.experimental.pallas import tpu_sc as plsc

N = 10000
E = 320000
D = 128
OUT = 128
G = 64
NEG = 0.01

NP = 10240
BLK = 1024
NB = NP // BLK
NS = 16
RPT = NP // NS
K = 128
SPT = 40
EPL = SPT * K
NLAUNCH = -(-((E + NS - 1) // NS) // EPL)
EPT = NLAUNCH * EPL
EPAD = EPT * NS
NCHUNK = EPT // K

_MESH = plsc.VectorSubcoreMesh(core_axis_name="c", subcore_axis_name="s")


def _sc_scatter_body(src1, dst1, src2, dst2, tab1, tab2, acc_in, out,
                     srcv, srcv2, dstv, rows, rows2, zbuf, acc, sem, sem2):
    c = lax.axis_index("c")
    s = lax.axis_index("s")
    for cc in (0, 1):
        @pl.when(c == cc)
        def _(cc=cc):
            def lb(t, carry):
                o2 = s * RPT + t * 32
                pltpu.sync_copy(acc_in.at[cc, pl.ds(o2, 32)], zbuf)
                pltpu.sync_copy(zbuf, acc.at[pl.ds(o2, 32)])
                return carry
            lax.fori_loop(0, RPT // 32, lb, 0)
    plsc.subcore_barrier()
    bufs = ((srcv, rows, sem), (srcv2, rows2, sem2))
    for cc, (src, dst, tab) in ((0, (src1, dst1, tab1)),
                                (1, (src2, dst2, tab2))):
        @pl.when(c == cc)
        def _(src=src, dst=dst, tab=tab):
            for p in range(SPT):
                sv, rv, sm = bufs[p % 2]
                pltpu.sync_copy(src.at[s, pl.ds(p * K, K)], sv)
                pltpu.sync_copy(dst.at[s, pl.ds(p * K, K)], dstv)
                pltpu.async_copy(tab.at[sv], rv, sm).wait()
                pltpu.sync_copy(rv, acc.at[dstv], add=True)
    plsc.subcore_barrier()
    for cc in (0, 1):
        @pl.when(c == cc)
        def _(cc=cc):
            def wbody(t, carry):
                o2 = s * RPT + t * 32
                pltpu.sync_copy(acc.at[pl.ds(o2, 32)], zbuf)
                pltpu.sync_copy(zbuf, out.at[cc, pl.ds(o2, 32)])
                return carry
            lax.fori_loop(0, RPT // 32, wbody, 0)


_sc_scatter = functools.partial(
    pl.kernel,
    out_type=jax.ShapeDtypeStruct((2, NP, OUT), jnp.float32),
    mesh=_MESH,
    scratch_types=[
        pltpu.VMEM((K,), jnp.int32),
        pltpu.VMEM((K,), jnp.int32),
        pltpu.VMEM((K,), jnp.int32),
        pltpu.VMEM((K, OUT), jnp.float32),
        pltpu.VMEM((K, OUT), jnp.float32),
        pltpu.VMEM((32, OUT), jnp.float32),
        pltpu.VMEM_SHARED((NP, OUT), jnp.float32),
        pltpu.SemaphoreType.DMA,
        pltpu.SemaphoreType.DMA,
    ],
)(_sc_scatter_body)


def _edge_pad(v):
    return jnp.concatenate(
        [v, jnp.full((EPAD - E,), N, jnp.int32)]).reshape(NS, NLAUNCH, EPL)


def _tc1_body(x1r, x2r, w1r, w2r, degr, h1r, hs1r, h2r, hs2r):
    for ci, (xr, wr, hr, hsr) in enumerate(((x1r, w1r, h1r, hs1r),
                                            (x2r, w2r, h2r, hs2r))):
        dinv = lax.rsqrt(degr[ci, :, 0:1] + 1.0)
        h = jnp.dot(xr[...], wr[...], preferred_element_type=jnp.float32)
        hr[...] = h
        hsr[...] = h * dinv


def _tc1(x1p, x2p, w1, w2, deg):
    row = pl.BlockSpec((BLK, D), lambda i: (i, 0))
    return pl.pallas_call(
        _tc1_body,
        grid=(NB,),
        in_specs=[row, row,
                  pl.BlockSpec((D, OUT), lambda i: (0, 0)),
                  pl.BlockSpec((D, OUT), lambda i: (0, 0)),
                  pl.BlockSpec((2, BLK, OUT), lambda i: (0, i, 0))],
        out_specs=[row, row, row, row],
        out_shape=[jax.ShapeDtypeStruct((NP, OUT), jnp.float32)] * 4,
    )(x1p, x2p, w1, w2, deg)


def _tc2_body(accr, h1r, h2r, degr, b1r, b2r, w1r, w2r,
              h1o, hs1o, h2o, hs2o):
    pid = pl.program_id(0)
    rid = lax.broadcasted_iota(jnp.int32, (BLK, 1), 0) + pid * BLK
    mask = rid < N
    for ci, (hr, br, wr, ho, hso) in enumerate(
            ((h1r, b1r, w1r, h1o, hs1o), (h2r, b2r, w2r, h2o, hs2o))):
        dinv = lax.rsqrt(degr[ci, :, 0:1] + 1.0)
        z = accr[ci] * dinv + (dinv * dinv) * hr[...] + br[...]
        z = jnp.where(z > 0, z, NEG * z)
        h2 = jnp.dot(z, wr[...], preferred_element_type=jnp.float32)
        ho[...] = h2
        hso[...] = jnp.where(mask, h2 * dinv, 0.0)


def _tc2(acc, h1, h2, deg, b1, b2, w1, w2):
    row = pl.BlockSpec((BLK, OUT), lambda i: (i, 0))
    return pl.pallas_call(
        _tc2_body,
        grid=(NB,),
        in_specs=[pl.BlockSpec((2, BLK, OUT), lambda i: (0, i, 0)),
                  row, row,
                  pl.BlockSpec((2, BLK, OUT), lambda i: (0, i, 0)),
                  pl.BlockSpec((1, OUT), lambda i: (0, 0)),
                  pl.BlockSpec((1, OUT), lambda i: (0, 0)),
                  pl.BlockSpec((OUT, OUT), lambda i: (0, 0)),
                  pl.BlockSpec((OUT, OUT), lambda i: (0, 0))],
        out_specs=[row, row, row, row],
        out_shape=[jax.ShapeDtypeStruct((NP, OUT), jnp.float32)] * 4,
    )(acc, h1, h2, deg, b1, b2, w1, w2)


def _tc3_body(accr, h1r, h2r, degr, b1r, b2r, bt1r, bt2r, psumr, cntr):
    pid = pl.program_id(0)

    @pl.when(pid == 0)
    def _():
        psumr[...] = jnp.zeros_like(psumr)
        cntr[...] = jnp.zeros_like(cntr)

    gio = lax.broadcasted_iota(jnp.int32, (G, BLK), 0)
    for ci, (hr, br, btr) in enumerate(((h1r, b1r, bt1r), (h2r, b2r, bt2r))):
        dinv = lax.rsqrt(degr[ci, :, 0:1] + 1.0)
        z = accr[ci] * dinv + (dinv * dinv) * hr[...] + br[...]
        z = jnp.where(z > 0, z, NEG * z)
        bt = btr[0, 0, :]
        oh = (gio == bt[None, :]).astype(jnp.float32)
        psumr[ci] = psumr[ci] + jnp.dot(oh, z,
                                        precision=lax.Precision.HIGHEST,
                                        preferred_element_type=jnp.float32)
        cntr[ci] = cntr[ci] + jnp.sum(oh, axis=1, keepdims=True)


def _tc3(acc, h1, h2, deg, b1, b2, bt1, bt2):
    row = pl.BlockSpec((BLK, OUT), lambda i: (i, 0))
    return pl.pallas_call(
        _tc3_body,
        grid=(NB,),
        in_specs=[pl.BlockSpec((2, BLK, OUT), lambda i: (0, i, 0)),
                  row, row,
                  pl.BlockSpec((2, BLK, OUT), lambda i: (0, i, 0)),
                  pl.BlockSpec((1, OUT), lambda i: (0, 0)),
                  pl.BlockSpec((1, OUT), lambda i: (0, 0)),
                  pl.BlockSpec((1, 1, BLK), lambda i: (i, 0, 0)),
                  pl.BlockSpec((1, 1, BLK), lambda i: (i, 0, 0))],
        out_specs=[pl.BlockSpec((2, G, OUT), lambda i: (0, 0, 0)),
                   pl.BlockSpec((2, G, OUT), lambda i: (0, 0, 0))],
        out_shape=[jax.ShapeDtypeStruct((2, G, OUT), jnp.float32)] * 2,
    )(acc, h1, h2, deg, b1, b2, bt1, bt2)


def _tc4_body(psumr, cntr, wl1r, bl1r, wl2r, bl2r, outr):
    p1 = psumr[0] / jnp.maximum(cntr[0], 1.0)
    p2 = psumr[1] / jnp.maximum(cntr[1], 1.0)
    hh = (jnp.dot(p1, wl1r[0:OUT, :], preferred_element_type=jnp.float32)
          + jnp.dot(p2, wl1r[OUT:2 * OUT, :],
                    preferred_element_type=jnp.float32)
          + bl1r[...])
    hh = jnp.maximum(hh, 0.0)
    outr[...] = jnp.dot(hh, wl2r[...],
                        preferred_element_type=jnp.float32) + bl2r[...]


def _tc4(psum, cnt, wl1, bl1, wl2p, bl2):
    return pl.pallas_call(
        _tc4_body,
        out_shape=jax.ShapeDtypeStruct((G, OUT), jnp.float32),
    )(psum, cnt, wl1, bl1, wl2p, bl2)


def kernel(x1, x2, edge_index1, edge_index2, batch1, batch2,
           W1a, b1a, W1b, b1b, W2a, b2a, W2b, b2b, Wl1, bl1, Wl2, bl2):
    f32 = jnp.float32
    x1p = jnp.pad(x1, ((0, NP - N), (0, 0)))
    x2p = jnp.pad(x2, ((0, NP - N), (0, 0)))
    src1 = _edge_pad(edge_index1[0])
    dst1 = _edge_pad(edge_index1[1])
    src2 = _edge_pad(edge_index2[0])
    dst2 = _edge_pad(edge_index2[1])
    bt1 = jnp.pad(batch1, (0, NP - N), constant_values=G).reshape(NB, 1, BLK)
    bt2 = jnp.pad(batch2, (0, NP - N), constant_values=G).reshape(NB, 1, BLK)
    wl2p = jnp.pad(Wl2, ((0, 0), (0, OUT - 1)))
    bl2r = jnp.broadcast_to(bl2.reshape(1, 1), (1, OUT))

    def scatter(tab1, tab2):
        acc = jnp.zeros((2, NP, OUT), f32)
        for j in range(NLAUNCH):
            acc = _sc_scatter(src1[:, j], dst1[:, j], src2[:, j], dst2[:, j],
                              tab1, tab2, acc)
        return acc

    ones_tab = jnp.ones((NP, OUT), f32)
    deg = scatter(ones_tab, ones_tab)
    h1, hs1, h2, hs2 = _tc1(x1p, x2p, W1a, W2a, deg)
    acc = scatter(hs1, hs2)
    h1b, hs1b, h2b, hs2b = _tc2(acc, h1, h2, deg,
                                b1a.reshape(1, OUT), b2a.reshape(1, OUT),
                                W1b, W2b)
    acc2 = scatter(hs1b, hs2b)
    psum, cnt = _tc3(acc2, h1b, h2b, deg,
                     b1b.reshape(1, OUT), b2b.reshape(1, OUT), bt1, bt2)
    outp = _tc4(psum, cnt, Wl1, bl1.reshape(1, OUT), wl2p, bl2r)
    return outp[:, 0:1]
